# baseline (speedup 1.0000x reference)
import functools
import math

import jax
import jax.numpy as jnp
from jax import lax
from jax.experimental import pallas as pl
from jax.experimental.pallas import tpu as pltpu
from jax.experimental.pallas import tpu_sc as plsc

B, S, H, G, K = 16, 1, 16, 1, 2048
DIM, TAIL = 512, 64
D = DIM + TAIL
SKV = 8192
SM_SCALE = 1.0 / math.sqrt(D)

NC, NS = 2, 16
NW = NC * NS
ROWS = B * K
RPW = ROWS // NW
CHUNK = 64
NCHUNK = RPW // CHUNK
LANES = 16


def _sc_gather(kv_flat, idx_flat):
    mesh = plsc.VectorSubcoreMesh(core_axis_name="c", subcore_axis_name="s")

    @functools.partial(
        pl.kernel,
        out_type=jax.ShapeDtypeStruct((ROWS, D), jnp.float32),
        mesh=mesh,
        scratch_types=[
            pltpu.VMEM((RPW,), jnp.int32),
            pltpu.VMEM((2, CHUNK, D), jnp.float32),
            pltpu.SemaphoreType.DMA,
            pltpu.SemaphoreType.DMA,
        ],
        compiler_params=pltpu.CompilerParams(use_tc_tiling_on_sc=False),
    )
    def gather_kernel(kv_hbm, idx_hbm, out_hbm, idx_v, rows_v, gsem, wsem):
        wid = lax.axis_index("s") * NC + lax.axis_index("c")
        base = wid * RPW
        pltpu.sync_copy(idx_hbm.at[pl.ds(base, RPW)], idx_v)
        off = (base // K) * SKV
        for j in range(RPW // LANES):
            sl = pl.ds(j * LANES, LANES)
            idx_v[sl] = idx_v[sl] + off

        def g_copy(c, buf):
            return pltpu.make_async_copy(
                kv_hbm.at[idx_v.at[pl.ds(c * CHUNK, CHUNK)]],
                rows_v.at[buf], gsem)

        def w_copy(c, buf):
            return pltpu.make_async_copy(
                rows_v.at[buf],
                out_hbm.at[pl.ds(base + c * CHUNK, CHUNK)], wsem)

        g_copy(0, 0).start()
        for c in range(NCHUNK):
            buf = c % 2
            if c + 1 < NCHUNK:
                if c >= 1:
                    w_copy(c - 1, (c - 1) % 2).wait()
                g_copy(c + 1, (c + 1) % 2).start()
            g_copy(c, buf).wait()
            w_copy(c, buf).start()
        if NCHUNK >= 2:
            w_copy(NCHUNK - 2, (NCHUNK - 2) % 2).wait()
        w_copy(NCHUNK - 1, (NCHUNK - 1) % 2).wait()

    return gather_kernel(kv_flat, idx_flat)


def _attn_body(q_ref, g_ref, o_ref):
    qb = q_ref[0]
    kb = g_ref[0]
    s = lax.dot_general(qb, kb, (((1,), (1,)), ((), ())),
                        preferred_element_type=jnp.float32) * SM_SCALE
    m = jnp.max(s, axis=1, keepdims=True)
    e = jnp.exp(s - m)
    den = jnp.sum(e, axis=1, keepdims=True)
    o = lax.dot_general(e, kb[:, :DIM], (((1,), (0,)), ((), ())),
                        preferred_element_type=jnp.float32)
    o_ref[0] = o / den


def _tc_attention(q3, gathered):
    return pl.pallas_call(
        _attn_body,
        grid=(B,),
        in_specs=[
            pl.BlockSpec((1, H, D), lambda b: (b, 0, 0)),
            pl.BlockSpec((1, K, D), lambda b: (b, 0, 0)),
        ],
        out_specs=pl.BlockSpec((1, H, DIM), lambda b: (b, 0, 0)),
        out_shape=jax.ShapeDtypeStruct((B, H, DIM), jnp.float32),
    )(q3, gathered)


def kernel(q, kv, indices):
    kv_flat = kv.reshape(B * SKV, D)
    idx_flat = indices.reshape(ROWS)
    gathered = _sc_gather(kv_flat, idx_flat)
    out = _tc_attention(q.reshape(B, H, D), gathered.reshape(B, K, D))
    return out.reshape(B, S, H, DIM)


# speedup vs baseline: 6.5589x; 6.5589x over previous
import functools
import math

import jax
import jax.numpy as jnp
from jax import lax
from jax.experimental import pallas as pl
from jax.experimental.pallas import tpu as pltpu
from jax.experimental.pallas import tpu_sc as plsc

B, S, H, G, K = 16, 1, 16, 1, 2048
DIM, TAIL = 512, 64
D = DIM + TAIL
SKV = 8192
SM_SCALE = 1.0 / math.sqrt(D)

NC, NS = 2, 16
LANES = 16


def _sc_histogram(idx2d):
    mesh = plsc.VectorSubcoreMesh(core_axis_name="c", subcore_axis_name="s")

    @functools.partial(
        pl.kernel,
        out_type=jax.ShapeDtypeStruct((B, 1, SKV), jnp.float32),
        mesh=mesh,
        scratch_types=[
            pltpu.VMEM((K,), jnp.int32),
            pltpu.VMEM((SKV,), jnp.float32),
        ],
        compiler_params=pltpu.CompilerParams(
            use_tc_tiling_on_sc=False, needs_layout_passes=False),
    )
    def hist_kernel(idx_hbm, out_hbm, idx_v, cnt_v):
        wid = lax.axis_index("s") * NC + lax.axis_index("c")

        @pl.when(wid < B)
        def _():
            pltpu.sync_copy(idx_hbm.at[wid], idx_v)
            zeros = jnp.zeros((LANES,), jnp.float32)
            for j in range(SKV // LANES):
                cnt_v[pl.ds(j * LANES, LANES)] = zeros
            ones = jnp.ones((LANES,), jnp.float32)
            for j in range(K // LANES):
                ids = idx_v[pl.ds(j * LANES, LANES)]
                plsc.addupdate_scatter(cnt_v, [ids], ones)
            pltpu.sync_copy(cnt_v, out_hbm.at[wid, 0])

    return hist_kernel(idx2d)


def _attn_body(q_ref, kvt_ref, cnt_ref, o_ref):
    qb = q_ref[0]
    kt = kvt_ref[0]
    cnt = cnt_ref[0]
    s = lax.dot_general(qb, kt, (((1,), (0,)), ((), ())),
                        preferred_element_type=jnp.float32) * SM_SCALE
    m = jnp.max(s, axis=1, keepdims=True)
    e = jnp.exp(s - m) * cnt
    den = jnp.sum(e, axis=1, keepdims=True)
    o = lax.dot_general(e, kt[:DIM, :], (((1,), (1,)), ((), ())),
                        preferred_element_type=jnp.float32)
    o_ref[0] = o / den


def _tc_attention(q3, kvt3, counts):
    return pl.pallas_call(
        _attn_body,
        grid=(B,),
        in_specs=[
            pl.BlockSpec((1, H, D), lambda b: (b, 0, 0)),
            pl.BlockSpec((1, D, SKV), lambda b: (b, 0, 0)),
            pl.BlockSpec((1, 1, SKV), lambda b: (b, 0, 0)),
        ],
        out_specs=pl.BlockSpec((1, H, DIM), lambda b: (b, 0, 0)),
        out_shape=jax.ShapeDtypeStruct((B, H, DIM), jnp.float32),
    )(q3, kvt3, counts)


def kernel(q, kv, indices):
    counts = _sc_histogram(indices.reshape(B, K))
    kvt3 = jnp.transpose(kv, (0, 2, 3, 1)).reshape(B, D, SKV)
    out = _tc_attention(q.reshape(B, H, D), kvt3, counts)
    return out.reshape(B, S, H, DIM)
